# Initial kernel scaffold; baseline (speedup 1.0000x reference)
import jax
import jax.numpy as jnp
from jax.experimental import pallas as pl


def kernel(h_V, h_E, center_id, batch_id, W_V, W_O, B1_w, B1_b, B2_w, B2_b, B3_w, B3_b):
    raise NotImplementedError("write your pallas kernel here")


# speedup vs baseline: 31.6020x; 31.6020x over previous
import functools

import jax
import jax.numpy as jnp
import numpy as np
from jax import lax
from jax.experimental import pallas as pl
from jax.experimental.pallas import tpu as pltpu
from jax.experimental.pallas import tpu_sc as plsc

_NC = 2
_NS = 16
_NW = _NC * _NS

_CH = 128


def _node_prep_body(hV_ref, W_ref, b_ref, out_ref):
    out_ref[...] = (
        jnp.dot(hV_ref[...], W_ref[...], preferred_element_type=jnp.float32)
        + b_ref[...]
    )


def _node_prep(h_V, B1V, b1, block):
    n, h = h_V.shape
    return pl.pallas_call(
        _node_prep_body,
        grid=(n // block,),
        in_specs=[
            pl.BlockSpec((block, h), lambda i: (i, 0)),
            pl.BlockSpec((h, h), lambda i: (0, 0)),
            pl.BlockSpec((1, h), lambda i: (0, 0)),
        ],
        out_specs=pl.BlockSpec((block, h), lambda i: (i, 0)),
        out_shape=jax.ShapeDtypeStruct((n, h), jnp.float32),
    )(h_V, B1V, b1)


def _sc_gather(table, cid):
    e = cid.shape[0]
    d = table.shape[1]
    n_chunks = e // _CH
    cpw = -(-n_chunks // _NW)
    mesh = plsc.VectorSubcoreMesh(core_axis_name="c", subcore_axis_name="s")

    @functools.partial(
        pl.kernel,
        out_type=jax.ShapeDtypeStruct((e, d), jnp.float32),
        mesh=mesh,
        scratch_types=[
            pltpu.VMEM((_CH,), jnp.int32),
            pltpu.VMEM((_CH, d), jnp.float32),
        ],
    )
    def k(table_hbm, cid_hbm, out_hbm, idx_v, rows_v):
        wid = lax.axis_index("s") * _NC + lax.axis_index("c")

        def body(i, carry):
            c = wid + i * _NW

            @pl.when(c < n_chunks)
            def _():
                base = c * _CH
                pltpu.sync_copy(cid_hbm.at[pl.ds(base, _CH)], idx_v)
                pltpu.sync_copy(table_hbm.at[idx_v], rows_v)
                pltpu.sync_copy(rows_v, out_hbm.at[pl.ds(base, _CH)])

            return carry

        lax.fori_loop(0, cpw, body, 0)

    return k(table, cid)


def _edge_body(hE_ref, G_ref, B1E_ref, B2_ref, b2_ref, B3p_ref, b3p_ref,
               WV_ref, sel_ref, P_ref, EB_ref):
    hE = hE_ref[...]
    h1 = jnp.maximum(
        G_ref[...] + jnp.dot(hE, B1E_ref[...], preferred_element_type=jnp.float32),
        0.0,
    )
    h2 = jnp.maximum(
        jnp.dot(h1, B2_ref[...], preferred_element_type=jnp.float32) + b2_ref[...],
        0.0,
    )
    logits = jnp.dot(h2, B3p_ref[...], preferred_element_type=jnp.float32) + b3p_ref[...]
    ex = jnp.exp(logits)
    eb = jnp.dot(ex, sel_ref[...], preferred_element_type=jnp.float32)
    v = jnp.dot(hE, WV_ref[...], preferred_element_type=jnp.float32)
    P_ref[...] = eb * v
    EB_ref[...] = eb


def _edge_compute(h_E, G, B1E, B2, b2, B3p, b3p, W_V, sel, block):
    e, din = h_E.shape
    h = G.shape[1]
    return pl.pallas_call(
        _edge_body,
        grid=(e // block,),
        in_specs=[
            pl.BlockSpec((block, din), lambda i: (i, 0)),
            pl.BlockSpec((block, h), lambda i: (i, 0)),
            pl.BlockSpec((din, h), lambda i: (0, 0)),
            pl.BlockSpec((h, h), lambda i: (0, 0)),
            pl.BlockSpec((1, h), lambda i: (0, 0)),
            pl.BlockSpec((h, h), lambda i: (0, 0)),
            pl.BlockSpec((1, h), lambda i: (0, 0)),
            pl.BlockSpec((din, h), lambda i: (0, 0)),
            pl.BlockSpec((h, h), lambda i: (0, 0)),
        ],
        out_specs=[
            pl.BlockSpec((block, h), lambda i: (i, 0)),
            pl.BlockSpec((block, h), lambda i: (i, 0)),
        ],
        out_shape=[
            jax.ShapeDtypeStruct((e, h), jnp.float32),
            jax.ShapeDtypeStruct((e, h), jnp.float32),
        ],
    )(h_E, G, B1E, B2, b2, B3p, b3p, W_V, sel)


def _sc_scatter_add(rows, cid, zeros, n_pad, rpt):
    e = cid.shape[0]
    d = rows.shape[1]
    n_chunks = e // _CH
    cpw = -(-n_chunks // _NW)
    mesh = plsc.VectorSubcoreMesh(core_axis_name="c", subcore_axis_name="s")

    @functools.partial(
        pl.kernel,
        out_type=jax.ShapeDtypeStruct((_NC, n_pad, d), jnp.float32),
        mesh=mesh,
        scratch_types=[
            pltpu.VMEM((_CH,), jnp.int32),
            pltpu.VMEM((_CH, d), jnp.float32),
            pltpu.VMEM_SHARED((n_pad, d), jnp.float32),
        ],
    )
    def k(rows_hbm, cid_hbm, z_hbm, out_hbm, idx_v, rows_v, acc_sh):
        cc = lax.axis_index("c")
        sid = lax.axis_index("s")
        wid = sid * _NC + cc

        pltpu.sync_copy(z_hbm.at[pl.ds(sid * rpt, rpt)],
                        acc_sh.at[pl.ds(sid * rpt, rpt)])
        plsc.subcore_barrier()

        def body(i, carry):
            c = wid + i * _NW

            @pl.when(c < n_chunks)
            def _():
                base = c * _CH
                pltpu.sync_copy(cid_hbm.at[pl.ds(base, _CH)], idx_v)
                pltpu.sync_copy(rows_hbm.at[pl.ds(base, _CH)], rows_v)
                pltpu.sync_copy(rows_v, acc_sh.at[idx_v], add=True)

            return carry

        lax.fori_loop(0, cpw, body, 0)
        plsc.subcore_barrier()

        pltpu.sync_copy(acc_sh.at[pl.ds(sid * rpt, rpt)],
                        out_hbm.at[cc, pl.ds(sid * rpt, rpt)])

    return k(rows, cid, zeros)


def _finish_body(SP_ref, SE_ref, WO_ref, out_ref):
    sp = SP_ref[0] + SP_ref[1]
    se = SE_ref[0] + SE_ref[1]
    h_agg = sp / jnp.where(se > 0.0, se, 1.0)
    out_ref[...] = jnp.dot(h_agg, WO_ref[...],
                           preferred_element_type=jnp.float32)


def _finish(SP, SE, W_O, n, block):
    h = W_O.shape[0]
    return pl.pallas_call(
        _finish_body,
        grid=(n // block,),
        in_specs=[
            pl.BlockSpec((2, block, h), lambda i: (0, i, 0)),
            pl.BlockSpec((2, block, h), lambda i: (0, i, 0)),
            pl.BlockSpec((h, h), lambda i: (0, 0)),
        ],
        out_specs=pl.BlockSpec((block, h), lambda i: (i, 0)),
        out_shape=jax.ShapeDtypeStruct((n, h), jnp.float32),
    )(SP, SE, W_O)


def kernel(h_V, h_E, center_id, batch_id, W_V, W_O,
           B1_w, B1_b, B2_w, B2_b, B3_w, B3_b):
    n, h = h_V.shape
    e, din = h_E.shape
    nh = B3_w.shape[1]
    dh = h // nh
    scale = 1.0 / np.sqrt(dh)

    B1V = B1_w[:h]
    B1E = B1_w[h:]
    b1 = B1_b.reshape(1, h)
    b2 = B2_b.reshape(1, h)
    B3p = jnp.zeros((h, h), jnp.float32).at[:, :nh].set(B3_w * scale)
    b3p = jnp.zeros((1, h), jnp.float32).at[0, :nh].set(B3_b * scale)
    sel_np = np.zeros((h, h), np.float32)
    for head in range(nh):
        sel_np[head, head * dh:(head + 1) * dh] = 1.0
    sel = jnp.asarray(sel_np)

    rpt = -(-n // (_NS * 8)) * 8
    n_pad = rpt * _NS
    zeros = jnp.zeros((n_pad, h), jnp.float32)

    A = _node_prep(h_V, B1V, b1, block=1000)
    G = _sc_gather(A, center_id)
    P, EB = _edge_compute(h_E, G, B1E, B2_w, b2, B3p, b3p, W_V, sel,
                          block=1000)
    SP = _sc_scatter_add(P, center_id, zeros, n_pad, rpt)
    SE = _sc_scatter_add(EB, center_id, zeros, n_pad, rpt)
    return _finish(SP, SE, W_O, n, block=1000)


# speedup vs baseline: 42.2945x; 1.3383x over previous
import functools

import jax
import jax.numpy as jnp
import numpy as np
from jax import lax
from jax.experimental import pallas as pl
from jax.experimental.pallas import tpu as pltpu
from jax.experimental.pallas import tpu_sc as plsc

_NC = 2
_NS = 16
_NW = _NC * _NS

_CH = 128


def _node_prep_body(hV_ref, W_ref, b_ref, out_ref):
    out_ref[...] = (
        jnp.dot(hV_ref[...], W_ref[...], preferred_element_type=jnp.float32)
        + b_ref[...]
    )


def _node_prep(h_V, B1V, b1, block):
    n, h = h_V.shape
    return pl.pallas_call(
        _node_prep_body,
        grid=(n // block,),
        in_specs=[
            pl.BlockSpec((block, h), lambda i: (i, 0)),
            pl.BlockSpec((h, h), lambda i: (0, 0)),
            pl.BlockSpec((1, h), lambda i: (0, 0)),
        ],
        out_specs=pl.BlockSpec((block, h), lambda i: (i, 0)),
        out_shape=jax.ShapeDtypeStruct((n, h), jnp.float32),
    )(h_V, B1V, b1)


def _sc_gather(table, cid):
    e = cid.shape[0]
    d = table.shape[1]
    n_chunks = e // _CH
    cpw = -(-n_chunks // _NW)
    niter = (cpw + 1) // 2
    mesh = plsc.VectorSubcoreMesh(core_axis_name="c", subcore_axis_name="s")

    @functools.partial(
        pl.kernel,
        out_type=jax.ShapeDtypeStruct((e, d), jnp.float32),
        mesh=mesh,
        scratch_types=[
            pltpu.VMEM((_CH,), jnp.int32),
            pltpu.VMEM((_CH,), jnp.int32),
            pltpu.VMEM((_CH, d), jnp.float32),
            pltpu.VMEM((_CH, d), jnp.float32),
            pltpu.SemaphoreType.DMA,
            pltpu.SemaphoreType.DMA,
            pltpu.SemaphoreType.DMA,
            pltpu.SemaphoreType.DMA,
        ],
    )
    def k(table_hbm, cid_hbm, out_hbm,
          idx0, idx1, rows0, rows1, si0, si1, sw0, sw1):
        wid = lax.axis_index("s") * _NC + lax.axis_index("c")
        idx_v = (idx0, idx1)
        rows_v = (rows0, rows1)
        sem_i = (si0, si1)
        sem_w = (sw0, sw1)

        def fire_idx(j, s):
            c = wid + j * _NW

            @pl.when(c < n_chunks)
            def _():
                pltpu.async_copy(cid_hbm.at[pl.ds(c * _CH, _CH)],
                                 idx_v[s], sem_i[s])

        def step(j, s):
            c = wid + j * _NW
            cm2 = c - 2 * _NW

            @pl.when((cm2 >= 0) & (cm2 < n_chunks))
            def _():
                pltpu.make_async_copy(
                    rows_v[s], out_hbm.at[pl.ds(cm2 * _CH, _CH)],
                    sem_w[s]).wait()

            @pl.when(c < n_chunks)
            def _():
                pltpu.make_async_copy(
                    cid_hbm.at[pl.ds(c * _CH, _CH)], idx_v[s],
                    sem_i[s]).wait()
                pltpu.sync_copy(table_hbm.at[idx_v[s]], rows_v[s])
                pltpu.async_copy(rows_v[s], out_hbm.at[pl.ds(c * _CH, _CH)],
                                 sem_w[s])

        fire_idx(0, 0)

        def body(jj, carry):
            j0 = 2 * jj
            fire_idx(j0 + 1, 1)
            step(j0, 0)
            fire_idx(j0 + 2, 0)
            step(j0 + 1, 1)
            return carry

        lax.fori_loop(0, niter, body, 0)

        for tail in (2 * niter - 2, 2 * niter - 1):
            c = wid + tail * _NW
            s = tail % 2

            @pl.when((c >= 0) & (c < n_chunks))
            def _():
                pltpu.make_async_copy(
                    rows_v[s], out_hbm.at[pl.ds(c * _CH, _CH)],
                    sem_w[s]).wait()

    return k(table, cid)


def _edge_body(hE_ref, G_ref, B1E_ref, B2_ref, b2_ref, B3p_ref, b3p_ref,
               WV_ref, sel_ref, P_ref, EB_ref):
    hE = hE_ref[...]
    h1 = jnp.maximum(
        G_ref[...] + jnp.dot(hE, B1E_ref[...], preferred_element_type=jnp.float32),
        0.0,
    )
    h2 = jnp.maximum(
        jnp.dot(h1, B2_ref[...], preferred_element_type=jnp.float32) + b2_ref[...],
        0.0,
    )
    logits = jnp.dot(h2, B3p_ref[...], preferred_element_type=jnp.float32) + b3p_ref[...]
    ex = jnp.exp(logits)
    eb = jnp.dot(ex, sel_ref[...], preferred_element_type=jnp.float32)
    v = jnp.dot(hE, WV_ref[...], preferred_element_type=jnp.float32)
    P_ref[...] = eb * v
    EB_ref[...] = eb


def _edge_compute(h_E, G, B1E, B2, b2, B3p, b3p, W_V, sel, block):
    e, din = h_E.shape
    h = G.shape[1]
    return pl.pallas_call(
        _edge_body,
        grid=(e // block,),
        in_specs=[
            pl.BlockSpec((block, din), lambda i: (i, 0)),
            pl.BlockSpec((block, h), lambda i: (i, 0)),
            pl.BlockSpec((din, h), lambda i: (0, 0)),
            pl.BlockSpec((h, h), lambda i: (0, 0)),
            pl.BlockSpec((1, h), lambda i: (0, 0)),
            pl.BlockSpec((h, h), lambda i: (0, 0)),
            pl.BlockSpec((1, h), lambda i: (0, 0)),
            pl.BlockSpec((din, h), lambda i: (0, 0)),
            pl.BlockSpec((h, h), lambda i: (0, 0)),
        ],
        out_specs=[
            pl.BlockSpec((block, h), lambda i: (i, 0)),
            pl.BlockSpec((block, h), lambda i: (i, 0)),
        ],
        out_shape=[
            jax.ShapeDtypeStruct((e, h), jnp.float32),
            jax.ShapeDtypeStruct((e, h), jnp.float32),
        ],
    )(h_E, G, B1E, B2, b2, B3p, b3p, W_V, sel)


def _sc_scatter_both(P, EB, cid, zeros, n_pad, rpt):
    e = cid.shape[0]
    d = P.shape[1]
    n_chunks = e // _CH
    cpw = -(-n_chunks // _NS)
    niter = (cpw + 1) // 2
    mesh = plsc.VectorSubcoreMesh(core_axis_name="c", subcore_axis_name="s")

    @functools.partial(
        pl.kernel,
        out_type=jax.ShapeDtypeStruct((_NC, n_pad, d), jnp.float32),
        mesh=mesh,
        scratch_types=[
            pltpu.VMEM((_CH,), jnp.int32),
            pltpu.VMEM((_CH,), jnp.int32),
            pltpu.VMEM((_CH, d), jnp.float32),
            pltpu.VMEM((_CH, d), jnp.float32),
            pltpu.VMEM_SHARED((n_pad, d), jnp.float32),
            pltpu.SemaphoreType.DMA,
            pltpu.SemaphoreType.DMA,
            pltpu.SemaphoreType.DMA,
            pltpu.SemaphoreType.DMA,
        ],
    )
    def k(P_hbm, EB_hbm, cid_hbm, z_hbm, out_hbm,
          idx0, idx1, rows0, rows1, acc_sh, si0, si1, sr0, sr1):
        cc = lax.axis_index("c")
        sid = lax.axis_index("s")
        idx_v = (idx0, idx1)
        rows_v = (rows0, rows1)
        sem_i = (si0, si1)
        sem_r = (sr0, sr1)

        pltpu.sync_copy(z_hbm.at[pl.ds(sid * rpt, rpt)],
                        acc_sh.at[pl.ds(sid * rpt, rpt)])
        plsc.subcore_barrier()

        def stream(rows_hbm):
            def fire(j, s):
                c = sid + j * _NS

                @pl.when(c < n_chunks)
                def _():
                    base = c * _CH
                    pltpu.async_copy(cid_hbm.at[pl.ds(base, _CH)],
                                     idx_v[s], sem_i[s])
                    pltpu.async_copy(rows_hbm.at[pl.ds(base, _CH)],
                                     rows_v[s], sem_r[s])

            def step(j, s):
                c = sid + j * _NS

                @pl.when(c < n_chunks)
                def _():
                    base = c * _CH
                    pltpu.make_async_copy(cid_hbm.at[pl.ds(base, _CH)],
                                          idx_v[s], sem_i[s]).wait()
                    pltpu.make_async_copy(rows_hbm.at[pl.ds(base, _CH)],
                                          rows_v[s], sem_r[s]).wait()
                    pltpu.sync_copy(rows_v[s], acc_sh.at[idx_v[s]], add=True)

            fire(0, 0)

            def body(jj, carry):
                j0 = 2 * jj
                fire(j0 + 1, 1)
                step(j0, 0)
                fire(j0 + 2, 0)
                step(j0 + 1, 1)
                return carry

            lax.fori_loop(0, niter, body, 0)

        @pl.when(cc == 0)
        def _():
            stream(P_hbm)

        @pl.when(cc == 1)
        def _():
            stream(EB_hbm)

        plsc.subcore_barrier()
        pltpu.sync_copy(acc_sh.at[pl.ds(sid * rpt, rpt)],
                        out_hbm.at[cc, pl.ds(sid * rpt, rpt)])

    return k(P, EB, cid, zeros)


def _finish_body(SP_ref, SE_ref, WO_ref, out_ref):
    se = SE_ref[...]
    h_agg = SP_ref[...] / jnp.where(se > 0.0, se, 1.0)
    out_ref[...] = jnp.dot(h_agg, WO_ref[...],
                           preferred_element_type=jnp.float32)


def _finish(S, W_O, n, block):
    h = W_O.shape[0]
    return pl.pallas_call(
        _finish_body,
        grid=(n // block,),
        in_specs=[
            pl.BlockSpec((None, block, h), lambda i: (0, i, 0)),
            pl.BlockSpec((None, block, h), lambda i: (1, i, 0)),
            pl.BlockSpec((h, h), lambda i: (0, 0)),
        ],
        out_specs=pl.BlockSpec((block, h), lambda i: (i, 0)),
        out_shape=jax.ShapeDtypeStruct((n, h), jnp.float32),
    )(S, S, W_O)


def kernel(h_V, h_E, center_id, batch_id, W_V, W_O,
           B1_w, B1_b, B2_w, B2_b, B3_w, B3_b):
    n, h = h_V.shape
    e, din = h_E.shape
    nh = B3_w.shape[1]
    dh = h // nh
    scale = 1.0 / np.sqrt(dh)

    B1V = B1_w[:h]
    B1E = B1_w[h:]
    b1 = B1_b.reshape(1, h)
    b2 = B2_b.reshape(1, h)
    B3p = jnp.zeros((h, h), jnp.float32).at[:, :nh].set(B3_w * scale)
    b3p = jnp.zeros((1, h), jnp.float32).at[0, :nh].set(B3_b * scale)
    sel_np = np.zeros((h, h), np.float32)
    for head in range(nh):
        sel_np[head, head * dh:(head + 1) * dh] = 1.0
    sel = jnp.asarray(sel_np)

    rpt = -(-n // (_NS * 8)) * 8
    n_pad = rpt * _NS
    zeros = jnp.zeros((n_pad, h), jnp.float32)

    A = _node_prep(h_V, B1V, b1, block=1000)
    G = _sc_gather(A, center_id)
    P, EB = _edge_compute(h_E, G, B1E, B2_w, b2, B3p, b3p, W_V, sel,
                          block=1000)
    S = _sc_scatter_both(P, EB, center_id, zeros, n_pad, rpt)
    return _finish(S, W_O, n, block=1000)


# speedup vs baseline: 48.9141x; 1.1565x over previous
import functools

import jax
import jax.numpy as jnp
import numpy as np
from jax import lax
from jax.experimental import pallas as pl
from jax.experimental.pallas import tpu as pltpu
from jax.experimental.pallas import tpu_sc as plsc

_NC = 2
_NS = 16
_NW = _NC * _NS

_CH = 128


def _node_prep_body(hV_ref, W_ref, b_ref, out_ref):
    out_ref[...] = (
        jnp.dot(hV_ref[...], W_ref[...], preferred_element_type=jnp.float32)
        + b_ref[...]
    )


def _node_prep(h_V, B1V, b1, block):
    n, h = h_V.shape
    return pl.pallas_call(
        _node_prep_body,
        grid=(n // block,),
        in_specs=[
            pl.BlockSpec((block, h), lambda i: (i, 0)),
            pl.BlockSpec((h, h), lambda i: (0, 0)),
            pl.BlockSpec((1, h), lambda i: (0, 0)),
        ],
        out_specs=pl.BlockSpec((block, h), lambda i: (i, 0)),
        out_shape=jax.ShapeDtypeStruct((n, h), jnp.float32),
    )(h_V, B1V, b1)


_CH2 = 256


def _sc_gather(table, cid):
    e = cid.shape[0]
    d = table.shape[1]
    n_chunks = e // _CH2
    cpw = -(-n_chunks // _NW)
    niter = (cpw + 1) // 2
    mesh = plsc.VectorSubcoreMesh(core_axis_name="c", subcore_axis_name="s")

    @functools.partial(
        pl.kernel,
        out_type=jax.ShapeDtypeStruct((e, d), jnp.float32),
        mesh=mesh,
        scratch_types=[
            pltpu.VMEM((_CH2,), jnp.int32),
            pltpu.VMEM((_CH2,), jnp.int32),
            pltpu.VMEM((_CH2, d), jnp.float32),
            pltpu.VMEM((_CH2, d), jnp.float32),
            pltpu.SemaphoreType.DMA,
            pltpu.SemaphoreType.DMA,
            pltpu.SemaphoreType.DMA,
            pltpu.SemaphoreType.DMA,
            pltpu.SemaphoreType.DMA,
            pltpu.SemaphoreType.DMA,
        ],
    )
    def k(table_hbm, cid_hbm, out_hbm,
          idx0, idx1, rows0, rows1, si0, si1, sg0, sg1, sw0, sw1):
        wid = lax.axis_index("s") * _NC + lax.axis_index("c")
        idx_v = (idx0, idx1)
        rows_v = (rows0, rows1)
        sem_i = (si0, si1)
        sem_g = (sg0, sg1)
        sem_w = (sw0, sw1)

        def fire_idx(j, s):
            c = wid + j * _NW

            @pl.when(c < n_chunks)
            def _():
                pltpu.async_copy(cid_hbm.at[pl.ds(c * _CH2, _CH2)],
                                 idx_v[s], sem_i[s])

        def wait_wb(j, s):
            c = wid + j * _NW

            @pl.when((c >= 0) & (c < n_chunks))
            def _():
                pltpu.make_async_copy(
                    rows_v[s], out_hbm.at[pl.ds(c * _CH2, _CH2)],
                    sem_w[s]).wait()

        def finalize(j, s):
            c = wid + j * _NW

            @pl.when((c >= 0) & (c < n_chunks))
            def _():
                for q in range(2):
                    pltpu.make_async_copy(
                        table_hbm.at[idx_v[s].at[pl.ds(q * _CH, _CH)]],
                        rows_v[s].at[pl.ds(q * _CH, _CH)], sem_g[s]).wait()
                pltpu.async_copy(rows_v[s], out_hbm.at[pl.ds(c * _CH2, _CH2)],
                                 sem_w[s])

        def half(j, s, o):
            c = wid + j * _NW

            @pl.when(c < n_chunks)
            def _():
                pltpu.make_async_copy(cid_hbm.at[pl.ds(c * _CH2, _CH2)],
                                      idx_v[s], sem_i[s]).wait()
                for q in range(2):
                    pltpu.async_copy(
                        table_hbm.at[idx_v[s].at[pl.ds(q * _CH, _CH)]],
                        rows_v[s].at[pl.ds(q * _CH, _CH)], sem_g[s])

            finalize(j - 1, o)
            fire_idx(j + 1, o)

        fire_idx(0, 0)

        def body(jj, carry):
            j0 = 2 * jj
            wait_wb(j0 - 2, 0)
            half(j0, 0, 1)
            wait_wb(j0 - 1, 1)
            half(j0 + 1, 1, 0)
            return carry

        lax.fori_loop(0, niter, body, 0)

        finalize(2 * niter - 1, (2 * niter - 1) % 2)
        wait_wb(2 * niter - 2, 0)
        wait_wb(2 * niter - 1, 1)

    return k(table, cid)


def _edge_body(hE_ref, G_ref, B1E_ref, B2_ref, b2_ref, B3p_ref, b3p_ref,
               WV_ref, sel_ref, P_ref, EB_ref):
    hE = hE_ref[...]
    h1 = jnp.maximum(
        G_ref[...] + jnp.dot(hE, B1E_ref[...], preferred_element_type=jnp.float32),
        0.0,
    )
    h2 = jnp.maximum(
        jnp.dot(h1, B2_ref[...], preferred_element_type=jnp.float32) + b2_ref[...],
        0.0,
    )
    logits = jnp.dot(h2, B3p_ref[...], preferred_element_type=jnp.float32) + b3p_ref[...]
    ex = jnp.exp(logits)
    eb = jnp.dot(ex, sel_ref[...], preferred_element_type=jnp.float32)
    v = jnp.dot(hE, WV_ref[...], preferred_element_type=jnp.float32)
    P_ref[...] = eb * v
    EB_ref[...] = eb


def _edge_compute(h_E, G, B1E, B2, b2, B3p, b3p, W_V, sel, block):
    e, din = h_E.shape
    h = G.shape[1]
    return pl.pallas_call(
        _edge_body,
        grid=(e // block,),
        in_specs=[
            pl.BlockSpec((block, din), lambda i: (i, 0)),
            pl.BlockSpec((block, h), lambda i: (i, 0)),
            pl.BlockSpec((din, h), lambda i: (0, 0)),
            pl.BlockSpec((h, h), lambda i: (0, 0)),
            pl.BlockSpec((1, h), lambda i: (0, 0)),
            pl.BlockSpec((h, h), lambda i: (0, 0)),
            pl.BlockSpec((1, h), lambda i: (0, 0)),
            pl.BlockSpec((din, h), lambda i: (0, 0)),
            pl.BlockSpec((h, h), lambda i: (0, 0)),
        ],
        out_specs=[
            pl.BlockSpec((block, h), lambda i: (i, 0)),
            pl.BlockSpec((block, h), lambda i: (i, 0)),
        ],
        out_shape=[
            jax.ShapeDtypeStruct((e, h), jnp.float32),
            jax.ShapeDtypeStruct((e, h), jnp.float32),
        ],
    )(h_E, G, B1E, B2, b2, B3p, b3p, W_V, sel)


def _sc_scatter_both(P, EB, cid, zeros, n_pad, rpt):
    e = cid.shape[0]
    d = P.shape[1]
    n_chunks = e // _CH
    cpw = -(-n_chunks // _NS)
    niter = (cpw + 1) // 2
    mesh = plsc.VectorSubcoreMesh(core_axis_name="c", subcore_axis_name="s")

    @functools.partial(
        pl.kernel,
        out_type=jax.ShapeDtypeStruct((_NC, n_pad, d), jnp.float32),
        mesh=mesh,
        scratch_types=[
            pltpu.VMEM((_CH,), jnp.int32),
            pltpu.VMEM((_CH,), jnp.int32),
            pltpu.VMEM((_CH, d), jnp.float32),
            pltpu.VMEM((_CH, d), jnp.float32),
            pltpu.VMEM_SHARED((n_pad, d), jnp.float32),
            pltpu.SemaphoreType.DMA,
            pltpu.SemaphoreType.DMA,
            pltpu.SemaphoreType.DMA,
            pltpu.SemaphoreType.DMA,
            pltpu.SemaphoreType.DMA,
            pltpu.SemaphoreType.DMA,
        ],
    )
    def k(P_hbm, EB_hbm, cid_hbm, z_hbm, out_hbm,
          idx0, idx1, rows0, rows1, acc_sh,
          si0, si1, sr0, sr1, ss0, ss1):
        cc = lax.axis_index("c")
        sid = lax.axis_index("s")
        idx_v = (idx0, idx1)
        rows_v = (rows0, rows1)
        sem_i = (si0, si1)
        sem_r = (sr0, sr1)
        sem_s = (ss0, ss1)

        pltpu.sync_copy(z_hbm.at[pl.ds(sid * rpt, rpt)],
                        acc_sh.at[pl.ds(sid * rpt, rpt)])
        plsc.subcore_barrier()

        def stream(rows_hbm):
            def fire_loads(j, s):
                c = sid + j * _NS

                @pl.when(c < n_chunks)
                def _():
                    base = c * _CH
                    pltpu.async_copy(cid_hbm.at[pl.ds(base, _CH)],
                                     idx_v[s], sem_i[s])
                    pltpu.async_copy(rows_hbm.at[pl.ds(base, _CH)],
                                     rows_v[s], sem_r[s])

            def wait_scat(j, s):
                c = sid + j * _NS

                @pl.when((c >= 0) & (c < n_chunks))
                def _():
                    pltpu.make_async_copy(rows_v[s], acc_sh.at[idx_v[s]],
                                          sem_s[s]).wait()

            def half(j, s, o):
                c = sid + j * _NS

                @pl.when(c < n_chunks)
                def _():
                    base = c * _CH
                    pltpu.make_async_copy(cid_hbm.at[pl.ds(base, _CH)],
                                          idx_v[s], sem_i[s]).wait()
                    pltpu.make_async_copy(rows_hbm.at[pl.ds(base, _CH)],
                                          rows_v[s], sem_r[s]).wait()
                    pltpu.async_copy(rows_v[s], acc_sh.at[idx_v[s]],
                                     sem_s[s], add=True)

                wait_scat(j - 1, o)
                fire_loads(j + 1, o)

            fire_loads(0, 0)

            def body(jj, carry):
                j0 = 2 * jj
                half(j0, 0, 1)
                half(j0 + 1, 1, 0)
                return carry

            lax.fori_loop(0, niter, body, 0)
            wait_scat(2 * niter - 1, (2 * niter - 1) % 2)

        @pl.when(cc == 0)
        def _():
            stream(P_hbm)

        @pl.when(cc == 1)
        def _():
            stream(EB_hbm)

        plsc.subcore_barrier()
        pltpu.sync_copy(acc_sh.at[pl.ds(sid * rpt, rpt)],
                        out_hbm.at[cc, pl.ds(sid * rpt, rpt)])

    return k(P, EB, cid, zeros)


def _finish_body(SP_ref, SE_ref, WO_ref, out_ref):
    se = SE_ref[...]
    h_agg = SP_ref[...] / jnp.where(se > 0.0, se, 1.0)
    out_ref[...] = jnp.dot(h_agg, WO_ref[...],
                           preferred_element_type=jnp.float32)


def _finish(S, W_O, n, block):
    h = W_O.shape[0]
    return pl.pallas_call(
        _finish_body,
        grid=(n // block,),
        in_specs=[
            pl.BlockSpec((None, block, h), lambda i: (0, i, 0)),
            pl.BlockSpec((None, block, h), lambda i: (1, i, 0)),
            pl.BlockSpec((h, h), lambda i: (0, 0)),
        ],
        out_specs=pl.BlockSpec((block, h), lambda i: (i, 0)),
        out_shape=jax.ShapeDtypeStruct((n, h), jnp.float32),
    )(S, S, W_O)


def kernel(h_V, h_E, center_id, batch_id, W_V, W_O,
           B1_w, B1_b, B2_w, B2_b, B3_w, B3_b):
    n, h = h_V.shape
    e, din = h_E.shape
    nh = B3_w.shape[1]
    dh = h // nh
    scale = 1.0 / np.sqrt(dh)

    B1V = B1_w[:h]
    B1E = B1_w[h:]
    b1 = B1_b.reshape(1, h)
    b2 = B2_b.reshape(1, h)
    B3p = jnp.zeros((h, h), jnp.float32).at[:, :nh].set(B3_w * scale)
    b3p = jnp.zeros((1, h), jnp.float32).at[0, :nh].set(B3_b * scale)
    sel_np = np.zeros((h, h), np.float32)
    for head in range(nh):
        sel_np[head, head * dh:(head + 1) * dh] = 1.0
    sel = jnp.asarray(sel_np)

    rpt = -(-n // (_NS * 8)) * 8
    n_pad = rpt * _NS
    zeros = jnp.zeros((n_pad, h), jnp.float32)

    A = _node_prep(h_V, B1V, b1, block=1000)
    G = _sc_gather(A, center_id)
    P, EB = _edge_compute(h_E, G, B1E, B2_w, b2, B3p, b3p, W_V, sel,
                          block=2000)
    S = _sc_scatter_both(P, EB, center_id, zeros, n_pad, rpt)
    return _finish(S, W_O, n, block=1000)


# speedup vs baseline: 50.8685x; 1.0400x over previous
import functools

import jax
import jax.numpy as jnp
import numpy as np
from jax import lax
from jax.experimental import pallas as pl
from jax.experimental.pallas import tpu as pltpu
from jax.experimental.pallas import tpu_sc as plsc

_NC = 2
_NS = 16
_NW = _NC * _NS

_CH = 128


def _node_prep_body(hV_ref, W_ref, b_ref, out_ref):
    out_ref[...] = (
        jnp.dot(hV_ref[...].astype(jnp.bfloat16),
                W_ref[...].astype(jnp.bfloat16),
                preferred_element_type=jnp.float32)
        + b_ref[...]
    )


def _node_prep(h_V, B1V, b1, block):
    n, h = h_V.shape
    return pl.pallas_call(
        _node_prep_body,
        grid=(n // block,),
        in_specs=[
            pl.BlockSpec((block, h), lambda i: (i, 0)),
            pl.BlockSpec((h, h), lambda i: (0, 0)),
            pl.BlockSpec((1, h), lambda i: (0, 0)),
        ],
        out_specs=pl.BlockSpec((block, h), lambda i: (i, 0)),
        out_shape=jax.ShapeDtypeStruct((n, h), jnp.float32),
    )(h_V, B1V, b1)


_CH2 = 256


def _sc_gather(table, cid):
    e = cid.shape[0]
    d = table.shape[1]
    n_chunks = e // _CH2
    cpw = -(-n_chunks // _NW)
    niter = (cpw + 1) // 2
    mesh = plsc.VectorSubcoreMesh(core_axis_name="c", subcore_axis_name="s")

    @functools.partial(
        pl.kernel,
        out_type=jax.ShapeDtypeStruct((e, d), jnp.float32),
        mesh=mesh,
        scratch_types=[
            pltpu.VMEM((_CH2,), jnp.int32),
            pltpu.VMEM((_CH2,), jnp.int32),
            pltpu.VMEM((_CH2, d), jnp.float32),
            pltpu.VMEM((_CH2, d), jnp.float32),
            pltpu.SemaphoreType.DMA,
            pltpu.SemaphoreType.DMA,
            pltpu.SemaphoreType.DMA,
            pltpu.SemaphoreType.DMA,
            pltpu.SemaphoreType.DMA,
            pltpu.SemaphoreType.DMA,
        ],
    )
    def k(table_hbm, cid_hbm, out_hbm,
          idx0, idx1, rows0, rows1, si0, si1, sg0, sg1, sw0, sw1):
        wid = lax.axis_index("s") * _NC + lax.axis_index("c")
        idx_v = (idx0, idx1)
        rows_v = (rows0, rows1)
        sem_i = (si0, si1)
        sem_g = (sg0, sg1)
        sem_w = (sw0, sw1)

        def fire_idx(j, s):
            c = wid + j * _NW

            @pl.when(c < n_chunks)
            def _():
                pltpu.async_copy(cid_hbm.at[pl.ds(c * _CH2, _CH2)],
                                 idx_v[s], sem_i[s])

        def wait_wb(j, s):
            c = wid + j * _NW

            @pl.when((c >= 0) & (c < n_chunks))
            def _():
                pltpu.make_async_copy(
                    rows_v[s], out_hbm.at[pl.ds(c * _CH2, _CH2)],
                    sem_w[s]).wait()

        def finalize(j, s):
            c = wid + j * _NW

            @pl.when((c >= 0) & (c < n_chunks))
            def _():
                for q in range(2):
                    pltpu.make_async_copy(
                        table_hbm.at[idx_v[s].at[pl.ds(q * _CH, _CH)]],
                        rows_v[s].at[pl.ds(q * _CH, _CH)], sem_g[s]).wait()
                pltpu.async_copy(rows_v[s], out_hbm.at[pl.ds(c * _CH2, _CH2)],
                                 sem_w[s])

        def half(j, s, o):
            c = wid + j * _NW

            @pl.when(c < n_chunks)
            def _():
                pltpu.make_async_copy(cid_hbm.at[pl.ds(c * _CH2, _CH2)],
                                      idx_v[s], sem_i[s]).wait()
                for q in range(2):
                    pltpu.async_copy(
                        table_hbm.at[idx_v[s].at[pl.ds(q * _CH, _CH)]],
                        rows_v[s].at[pl.ds(q * _CH, _CH)], sem_g[s])

            finalize(j - 1, o)
            fire_idx(j + 1, o)

        fire_idx(0, 0)

        def body(jj, carry):
            j0 = 2 * jj
            wait_wb(j0 - 2, 0)
            half(j0, 0, 1)
            wait_wb(j0 - 1, 1)
            half(j0 + 1, 1, 0)
            return carry

        lax.fori_loop(0, niter, body, 0)

        finalize(2 * niter - 1, (2 * niter - 1) % 2)
        wait_wb(2 * niter - 2, 0)
        wait_wb(2 * niter - 1, 1)

    return k(table, cid)


def _edge_body(hE_ref, G_ref, B1E_ref, B2_ref, b2_ref, B3p_ref, b3p_ref,
               WV_ref, sel_ref, P_ref, EB_ref):
    bf = jnp.bfloat16
    hE = hE_ref[...].astype(bf)
    h1 = jnp.maximum(
        G_ref[...] + jnp.dot(hE, B1E_ref[...].astype(bf),
                             preferred_element_type=jnp.float32),
        0.0,
    )
    h2 = jnp.maximum(
        jnp.dot(h1.astype(bf), B2_ref[...].astype(bf),
                preferred_element_type=jnp.float32) + b2_ref[...],
        0.0,
    )
    logits = jnp.dot(h2.astype(bf), B3p_ref[...].astype(bf),
                     preferred_element_type=jnp.float32) + b3p_ref[...]
    ex = jnp.exp(logits)
    eb = jnp.dot(ex.astype(bf), sel_ref[...].astype(bf),
                 preferred_element_type=jnp.float32)
    v = jnp.dot(hE, WV_ref[...].astype(bf), preferred_element_type=jnp.float32)
    P_ref[...] = eb * v
    EB_ref[...] = eb


def _edge_compute(h_E, G, B1E, B2, b2, B3p, b3p, W_V, sel, block):
    e, din = h_E.shape
    h = G.shape[1]
    return pl.pallas_call(
        _edge_body,
        grid=(e // block,),
        in_specs=[
            pl.BlockSpec((block, din), lambda i: (i, 0)),
            pl.BlockSpec((block, h), lambda i: (i, 0)),
            pl.BlockSpec((din, h), lambda i: (0, 0)),
            pl.BlockSpec((h, h), lambda i: (0, 0)),
            pl.BlockSpec((1, h), lambda i: (0, 0)),
            pl.BlockSpec((h, h), lambda i: (0, 0)),
            pl.BlockSpec((1, h), lambda i: (0, 0)),
            pl.BlockSpec((din, h), lambda i: (0, 0)),
            pl.BlockSpec((h, h), lambda i: (0, 0)),
        ],
        out_specs=[
            pl.BlockSpec((block, h), lambda i: (i, 0)),
            pl.BlockSpec((block, h), lambda i: (i, 0)),
        ],
        out_shape=[
            jax.ShapeDtypeStruct((e, h), jnp.float32),
            jax.ShapeDtypeStruct((e, h), jnp.float32),
        ],
    )(h_E, G, B1E, B2, b2, B3p, b3p, W_V, sel)


def _sc_scatter_both(P, EB, cid, zeros, n_pad, rpt):
    e = cid.shape[0]
    d = P.shape[1]
    n_chunks = e // _CH
    cpw = -(-n_chunks // _NS)
    niter = (cpw + 1) // 2
    mesh = plsc.VectorSubcoreMesh(core_axis_name="c", subcore_axis_name="s")

    @functools.partial(
        pl.kernel,
        out_type=jax.ShapeDtypeStruct((_NC, n_pad, d), jnp.float32),
        mesh=mesh,
        scratch_types=[
            pltpu.VMEM((_CH,), jnp.int32),
            pltpu.VMEM((_CH,), jnp.int32),
            pltpu.VMEM((_CH, d), jnp.float32),
            pltpu.VMEM((_CH, d), jnp.float32),
            pltpu.VMEM_SHARED((n_pad, d), jnp.float32),
            pltpu.SemaphoreType.DMA,
            pltpu.SemaphoreType.DMA,
            pltpu.SemaphoreType.DMA,
            pltpu.SemaphoreType.DMA,
        ],
    )
    def k(P_hbm, EB_hbm, cid_hbm, z_hbm, out_hbm,
          idx0, idx1, rows0, rows1, acc_sh,
          si0, si1, sr0, sr1):
        cc = lax.axis_index("c")
        sid = lax.axis_index("s")
        idx_v = (idx0, idx1)
        rows_v = (rows0, rows1)
        sem_i = (si0, si1)
        sem_r = (sr0, sr1)

        pltpu.sync_copy(z_hbm.at[pl.ds(sid * rpt, rpt)],
                        acc_sh.at[pl.ds(sid * rpt, rpt)])
        plsc.subcore_barrier()

        def stream(rows_hbm):
            def fire_loads(j, s):
                c = sid + j * _NS

                @pl.when(c < n_chunks)
                def _():
                    base = c * _CH
                    pltpu.async_copy(cid_hbm.at[pl.ds(base, _CH)],
                                     idx_v[s], sem_i[s])
                    pltpu.async_copy(rows_hbm.at[pl.ds(base, _CH)],
                                     rows_v[s], sem_r[s])

            def step(j, s, o):
                c = sid + j * _NS
                fire_loads(j + 1, o)

                @pl.when(c < n_chunks)
                def _():
                    base = c * _CH
                    pltpu.make_async_copy(cid_hbm.at[pl.ds(base, _CH)],
                                          idx_v[s], sem_i[s]).wait()
                    pltpu.make_async_copy(rows_hbm.at[pl.ds(base, _CH)],
                                          rows_v[s], sem_r[s]).wait()
                    pltpu.sync_copy(rows_v[s], acc_sh.at[idx_v[s]], add=True)

            fire_loads(0, 0)

            def body(jj, carry):
                j0 = 2 * jj
                step(j0, 0, 1)
                step(j0 + 1, 1, 0)
                return carry

            lax.fori_loop(0, niter, body, 0)

        @pl.when(cc == 0)
        def _():
            stream(P_hbm)

        @pl.when(cc == 1)
        def _():
            stream(EB_hbm)

        plsc.subcore_barrier()
        pltpu.sync_copy(acc_sh.at[pl.ds(sid * rpt, rpt)],
                        out_hbm.at[cc, pl.ds(sid * rpt, rpt)])

    return k(P, EB, cid, zeros)


def _finish_body(SP_ref, SE_ref, WO_ref, out_ref):
    se = SE_ref[...]
    h_agg = SP_ref[...] / jnp.where(se > 0.0, se, 1.0)
    out_ref[...] = jnp.dot(h_agg, WO_ref[...],
                           preferred_element_type=jnp.float32)


def _finish(S, W_O, n, block):
    h = W_O.shape[0]
    return pl.pallas_call(
        _finish_body,
        grid=(n // block,),
        in_specs=[
            pl.BlockSpec((None, block, h), lambda i: (0, i, 0)),
            pl.BlockSpec((None, block, h), lambda i: (1, i, 0)),
            pl.BlockSpec((h, h), lambda i: (0, 0)),
        ],
        out_specs=pl.BlockSpec((block, h), lambda i: (i, 0)),
        out_shape=jax.ShapeDtypeStruct((n, h), jnp.float32),
    )(S, S, W_O)


def kernel(h_V, h_E, center_id, batch_id, W_V, W_O,
           B1_w, B1_b, B2_w, B2_b, B3_w, B3_b):
    n, h = h_V.shape
    e, din = h_E.shape
    nh = B3_w.shape[1]
    dh = h // nh
    scale = 1.0 / np.sqrt(dh)

    B1V = B1_w[:h]
    B1E = B1_w[h:]
    b1 = B1_b.reshape(1, h)
    b2 = B2_b.reshape(1, h)
    B3p = jnp.zeros((h, h), jnp.float32).at[:, :nh].set(B3_w * scale)
    b3p = jnp.zeros((1, h), jnp.float32).at[0, :nh].set(B3_b * scale)
    sel_np = np.zeros((h, h), np.float32)
    for head in range(nh):
        sel_np[head, head * dh:(head + 1) * dh] = 1.0
    sel = jnp.asarray(sel_np)

    rpt = -(-n // (_NS * 8)) * 8
    n_pad = rpt * _NS
    zeros = jnp.zeros((n_pad, h), jnp.float32)

    A = _node_prep(h_V, B1V, b1, block=1000)
    G = _sc_gather(A, center_id)
    P, EB = _edge_compute(h_E, G, B1E, B2_w, b2, B3p, b3p, W_V, sel,
                          block=2000)
    S = _sc_scatter_both(P, EB, center_id, zeros, n_pad, rpt)
    return _finish(S, W_O, n, block=1000)


# speedup vs baseline: 52.4438x; 1.0310x over previous
import functools

import jax
import jax.numpy as jnp
import numpy as np
from jax import lax
from jax.experimental import pallas as pl
from jax.experimental.pallas import tpu as pltpu
from jax.experimental.pallas import tpu_sc as plsc

_NC = 2
_NS = 16
_NW = _NC * _NS

_CH = 128


def _node_prep_body(hV_ref, W_ref, b_ref, out_ref):
    out_ref[...] = (
        jnp.dot(hV_ref[...].astype(jnp.bfloat16),
                W_ref[...].astype(jnp.bfloat16),
                preferred_element_type=jnp.float32)
        + b_ref[...]
    )


def _node_prep(h_V, B1V, b1, block):
    n, h = h_V.shape
    return pl.pallas_call(
        _node_prep_body,
        grid=(n // block,),
        in_specs=[
            pl.BlockSpec((block, h), lambda i: (i, 0)),
            pl.BlockSpec((h, h), lambda i: (0, 0)),
            pl.BlockSpec((1, h), lambda i: (0, 0)),
        ],
        out_specs=pl.BlockSpec((block, h), lambda i: (i, 0)),
        out_shape=jax.ShapeDtypeStruct((n, h), jnp.float32),
    )(h_V, B1V, b1)


_CH2 = 256


def _sc_gather(table, cid, c0, n_chunks):
    d = table.shape[1]
    cpw = -(-n_chunks // _NW)
    niter = (cpw + 1) // 2
    mesh = plsc.VectorSubcoreMesh(core_axis_name="c", subcore_axis_name="s")

    @functools.partial(
        pl.kernel,
        out_type=jax.ShapeDtypeStruct((n_chunks * _CH2, d), jnp.float32),
        mesh=mesh,
        scratch_types=[
            pltpu.VMEM((_CH2,), jnp.int32),
            pltpu.VMEM((_CH2,), jnp.int32),
            pltpu.VMEM((_CH2, d), jnp.float32),
            pltpu.VMEM((_CH2, d), jnp.float32),
            pltpu.SemaphoreType.DMA,
            pltpu.SemaphoreType.DMA,
            pltpu.SemaphoreType.DMA,
            pltpu.SemaphoreType.DMA,
            pltpu.SemaphoreType.DMA,
            pltpu.SemaphoreType.DMA,
        ],
    )
    def k(table_hbm, cid_hbm, out_hbm,
          idx0, idx1, rows0, rows1, si0, si1, sg0, sg1, sw0, sw1):
        wid = lax.axis_index("s") * _NC + lax.axis_index("c")
        idx_v = (idx0, idx1)
        rows_v = (rows0, rows1)
        sem_i = (si0, si1)
        sem_g = (sg0, sg1)
        sem_w = (sw0, sw1)

        def fire_idx(j, s):
            c = wid + j * _NW

            @pl.when(c < n_chunks)
            def _():
                pltpu.async_copy(cid_hbm.at[pl.ds((c0 + c) * _CH2, _CH2)],
                                 idx_v[s], sem_i[s])

        def wait_wb(j, s):
            c = wid + j * _NW

            @pl.when((c >= 0) & (c < n_chunks))
            def _():
                pltpu.make_async_copy(
                    rows_v[s], out_hbm.at[pl.ds(c * _CH2, _CH2)],
                    sem_w[s]).wait()

        def finalize(j, s):
            c = wid + j * _NW

            @pl.when((c >= 0) & (c < n_chunks))
            def _():
                for q in range(2):
                    pltpu.make_async_copy(
                        table_hbm.at[idx_v[s].at[pl.ds(q * _CH, _CH)]],
                        rows_v[s].at[pl.ds(q * _CH, _CH)], sem_g[s]).wait()
                pltpu.async_copy(rows_v[s], out_hbm.at[pl.ds(c * _CH2, _CH2)],
                                 sem_w[s])

        def half(j, s, o):
            c = wid + j * _NW

            @pl.when(c < n_chunks)
            def _():
                pltpu.make_async_copy(cid_hbm.at[pl.ds((c0 + c) * _CH2, _CH2)],
                                      idx_v[s], sem_i[s]).wait()
                for q in range(2):
                    pltpu.async_copy(
                        table_hbm.at[idx_v[s].at[pl.ds(q * _CH, _CH)]],
                        rows_v[s].at[pl.ds(q * _CH, _CH)], sem_g[s])

            finalize(j - 1, o)
            fire_idx(j + 1, o)

        fire_idx(0, 0)

        def body(jj, carry):
            j0 = 2 * jj
            wait_wb(j0 - 2, 0)
            half(j0, 0, 1)
            wait_wb(j0 - 1, 1)
            half(j0 + 1, 1, 0)
            return carry

        lax.fori_loop(0, niter, body, 0)

        finalize(2 * niter - 1, (2 * niter - 1) % 2)
        wait_wb(2 * niter - 2, 0)
        wait_wb(2 * niter - 1, 1)

    return k(table, cid)


def _edge_body(hE_ref, G_ref, B1E_ref, B2_ref, b2_ref, B3p_ref, b3p_ref,
               WV_ref, sel_ref, P_ref, EB_ref):
    bf = jnp.bfloat16
    hE = hE_ref[...].astype(bf)
    h1 = jnp.maximum(
        G_ref[...] + jnp.dot(hE, B1E_ref[...].astype(bf),
                             preferred_element_type=jnp.float32),
        0.0,
    )
    h2 = jnp.maximum(
        jnp.dot(h1.astype(bf), B2_ref[...].astype(bf),
                preferred_element_type=jnp.float32) + b2_ref[...],
        0.0,
    )
    logits = jnp.dot(h2.astype(bf), B3p_ref[...].astype(bf),
                     preferred_element_type=jnp.float32) + b3p_ref[...]
    ex = jnp.exp(logits)
    eb = jnp.dot(ex.astype(bf), sel_ref[...].astype(bf),
                 preferred_element_type=jnp.float32)
    v = jnp.dot(hE, WV_ref[...].astype(bf), preferred_element_type=jnp.float32)
    P_ref[...] = eb * v
    EB_ref[...] = eb


def _edge_compute(h_E, G, B1E, B2, b2, B3p, b3p, W_V, sel, block, blk0):
    din = h_E.shape[1]
    e, h = G.shape
    return pl.pallas_call(
        _edge_body,
        grid=(e // block,),
        in_specs=[
            pl.BlockSpec((block, din), lambda i: (i + blk0, 0)),
            pl.BlockSpec((block, h), lambda i: (i, 0)),
            pl.BlockSpec((din, h), lambda i: (0, 0)),
            pl.BlockSpec((h, h), lambda i: (0, 0)),
            pl.BlockSpec((1, h), lambda i: (0, 0)),
            pl.BlockSpec((h, h), lambda i: (0, 0)),
            pl.BlockSpec((1, h), lambda i: (0, 0)),
            pl.BlockSpec((din, h), lambda i: (0, 0)),
            pl.BlockSpec((h, h), lambda i: (0, 0)),
        ],
        out_specs=[
            pl.BlockSpec((block, h), lambda i: (i, 0)),
            pl.BlockSpec((block, h), lambda i: (i, 0)),
        ],
        out_shape=[
            jax.ShapeDtypeStruct((e, h), jnp.float32),
            jax.ShapeDtypeStruct((e, h), jnp.float32),
        ],
    )(h_E, G, B1E, B2, b2, B3p, b3p, W_V, sel)


def _sc_scatter_both(P, EB, cid, init, n_pad, rpt, c0):
    d = P.shape[1]
    n_chunks = P.shape[0] // _CH
    cpw = -(-n_chunks // _NS)
    niter = (cpw + 1) // 2
    mesh = plsc.VectorSubcoreMesh(core_axis_name="c", subcore_axis_name="s")

    @functools.partial(
        pl.kernel,
        out_type=jax.ShapeDtypeStruct((_NC, n_pad, d), jnp.float32),
        mesh=mesh,
        scratch_types=[
            pltpu.VMEM((_CH,), jnp.int32),
            pltpu.VMEM((_CH,), jnp.int32),
            pltpu.VMEM((_CH, d), jnp.float32),
            pltpu.VMEM((_CH, d), jnp.float32),
            pltpu.VMEM_SHARED((n_pad, d), jnp.float32),
            pltpu.SemaphoreType.DMA,
            pltpu.SemaphoreType.DMA,
            pltpu.SemaphoreType.DMA,
            pltpu.SemaphoreType.DMA,
        ],
    )
    def k(P_hbm, EB_hbm, cid_hbm, z_hbm, out_hbm,
          idx0, idx1, rows0, rows1, acc_sh,
          si0, si1, sr0, sr1):
        cc = lax.axis_index("c")
        sid = lax.axis_index("s")
        idx_v = (idx0, idx1)
        rows_v = (rows0, rows1)
        sem_i = (si0, si1)
        sem_r = (sr0, sr1)

        pltpu.sync_copy(z_hbm.at[cc, pl.ds(sid * rpt, rpt)],
                        acc_sh.at[pl.ds(sid * rpt, rpt)])
        plsc.subcore_barrier()

        def stream(rows_hbm):
            def fire_loads(j, s):
                c = sid + j * _NS

                @pl.when(c < n_chunks)
                def _():
                    pltpu.async_copy(cid_hbm.at[pl.ds((c0 + c) * _CH, _CH)],
                                     idx_v[s], sem_i[s])
                    pltpu.async_copy(rows_hbm.at[pl.ds(c * _CH, _CH)],
                                     rows_v[s], sem_r[s])

            def step(j, s, o):
                c = sid + j * _NS
                fire_loads(j + 1, o)

                @pl.when(c < n_chunks)
                def _():
                    pltpu.make_async_copy(cid_hbm.at[pl.ds((c0 + c) * _CH, _CH)],
                                          idx_v[s], sem_i[s]).wait()
                    pltpu.make_async_copy(rows_hbm.at[pl.ds(c * _CH, _CH)],
                                          rows_v[s], sem_r[s]).wait()
                    pltpu.sync_copy(rows_v[s], acc_sh.at[idx_v[s]], add=True)

            fire_loads(0, 0)

            def body(jj, carry):
                j0 = 2 * jj
                step(j0, 0, 1)
                step(j0 + 1, 1, 0)
                return carry

            lax.fori_loop(0, niter, body, 0)

        @pl.when(cc == 0)
        def _():
            stream(P_hbm)

        @pl.when(cc == 1)
        def _():
            stream(EB_hbm)

        plsc.subcore_barrier()
        pltpu.sync_copy(acc_sh.at[pl.ds(sid * rpt, rpt)],
                        out_hbm.at[cc, pl.ds(sid * rpt, rpt)])

    return k(P, EB, cid, init)


def _finish_body(SP_ref, SE_ref, WO_ref, out_ref):
    se = SE_ref[...]
    h_agg = SP_ref[...] / jnp.where(se > 0.0, se, 1.0)
    out_ref[...] = jnp.dot(h_agg, WO_ref[...],
                           preferred_element_type=jnp.float32)


def _finish(S, W_O, n, block):
    h = W_O.shape[0]
    return pl.pallas_call(
        _finish_body,
        grid=(n // block,),
        in_specs=[
            pl.BlockSpec((None, block, h), lambda i: (0, i, 0)),
            pl.BlockSpec((None, block, h), lambda i: (1, i, 0)),
            pl.BlockSpec((h, h), lambda i: (0, 0)),
        ],
        out_specs=pl.BlockSpec((block, h), lambda i: (i, 0)),
        out_shape=jax.ShapeDtypeStruct((n, h), jnp.float32),
    )(S, S, W_O)


def kernel(h_V, h_E, center_id, batch_id, W_V, W_O,
           B1_w, B1_b, B2_w, B2_b, B3_w, B3_b):
    n, h = h_V.shape
    e, din = h_E.shape
    nh = B3_w.shape[1]
    dh = h // nh
    scale = 1.0 / np.sqrt(dh)

    B1V = B1_w[:h]
    B1E = B1_w[h:]
    b1 = B1_b.reshape(1, h)
    b2 = B2_b.reshape(1, h)
    B3p = jnp.zeros((h, h), jnp.float32).at[:, :nh].set(B3_w * scale)
    b3p = jnp.zeros((1, h), jnp.float32).at[0, :nh].set(B3_b * scale)
    sel_np = np.zeros((h, h), np.float32)
    for head in range(nh):
        sel_np[head, head * dh:(head + 1) * dh] = 1.0
    sel = jnp.asarray(sel_np)

    rpt = -(-n // (_NS * 8)) * 8
    n_pad = rpt * _NS
    zeros = jnp.zeros((_NC, n_pad, h), jnp.float32)

    blk = 1280
    nch1 = (e // _CH2 // 2) // 5 * 5
    e1 = nch1 * _CH2
    nch2 = e // _CH2 - nch1

    A = _node_prep(h_V, B1V, b1, block=1000)
    G1 = _sc_gather(A, center_id, 0, nch1)
    G2 = _sc_gather(A, center_id, nch1, nch2)
    P1, EB1 = _edge_compute(h_E, G1, B1E, B2_w, b2, B3p, b3p, W_V, sel,
                            block=blk, blk0=0)
    P2, EB2 = _edge_compute(h_E, G2, B1E, B2_w, b2, B3p, b3p, W_V, sel,
                            block=blk, blk0=e1 // blk)
    S1 = _sc_scatter_both(P1, EB1, center_id, zeros, n_pad, rpt, 0)
    S = _sc_scatter_both(P2, EB2, center_id, S1, n_pad, rpt, e1 // _CH)
    return _finish(S, W_O, n, block=1000)


# speedup vs baseline: 52.7047x; 1.0050x over previous
import functools

import jax
import jax.numpy as jnp
import numpy as np
from jax import lax
from jax.experimental import pallas as pl
from jax.experimental.pallas import tpu as pltpu
from jax.experimental.pallas import tpu_sc as plsc

_NC = 2
_NS = 16
_NW = _NC * _NS

_CH = 128


def _node_prep_body(hV_ref, W_ref, b_ref, out_ref):
    out_ref[...] = (
        jnp.dot(hV_ref[...].astype(jnp.bfloat16),
                W_ref[...].astype(jnp.bfloat16),
                preferred_element_type=jnp.float32)
        + b_ref[...]
    )


def _node_prep(h_V, B1V, b1, block):
    n, h = h_V.shape
    return pl.pallas_call(
        _node_prep_body,
        grid=(n // block,),
        in_specs=[
            pl.BlockSpec((block, h), lambda i: (i, 0)),
            pl.BlockSpec((h, h), lambda i: (0, 0)),
            pl.BlockSpec((1, h), lambda i: (0, 0)),
        ],
        out_specs=pl.BlockSpec((block, h), lambda i: (i, 0)),
        out_shape=jax.ShapeDtypeStruct((n, h), jnp.float32),
    )(h_V, B1V, b1)


_CH2 = 256


def _sc_gather(table, cid, c0, n_chunks):
    d = table.shape[1]
    cpw = -(-n_chunks // _NW)
    niter = (cpw + 1) // 2
    mesh = plsc.VectorSubcoreMesh(core_axis_name="c", subcore_axis_name="s")

    @functools.partial(
        pl.kernel,
        out_type=jax.ShapeDtypeStruct((n_chunks * _CH2, d), jnp.float32),
        mesh=mesh,
        scratch_types=[
            pltpu.VMEM((_CH2,), jnp.int32),
            pltpu.VMEM((_CH2,), jnp.int32),
            pltpu.VMEM((_CH2, d), jnp.float32),
            pltpu.VMEM((_CH2, d), jnp.float32),
            pltpu.SemaphoreType.DMA,
            pltpu.SemaphoreType.DMA,
            pltpu.SemaphoreType.DMA,
            pltpu.SemaphoreType.DMA,
            pltpu.SemaphoreType.DMA,
            pltpu.SemaphoreType.DMA,
        ],
    )
    def k(table_hbm, cid_hbm, out_hbm,
          idx0, idx1, rows0, rows1, si0, si1, sg0, sg1, sw0, sw1):
        wid = lax.axis_index("s") * _NC + lax.axis_index("c")
        idx_v = (idx0, idx1)
        rows_v = (rows0, rows1)
        sem_i = (si0, si1)
        sem_g = (sg0, sg1)
        sem_w = (sw0, sw1)

        def fire_idx(j, s):
            c = wid + j * _NW

            @pl.when(c < n_chunks)
            def _():
                pltpu.async_copy(cid_hbm.at[pl.ds((c0 + c) * _CH2, _CH2)],
                                 idx_v[s], sem_i[s])

        def wait_wb(j, s):
            c = wid + j * _NW

            @pl.when((c >= 0) & (c < n_chunks))
            def _():
                pltpu.make_async_copy(
                    rows_v[s], out_hbm.at[pl.ds(c * _CH2, _CH2)],
                    sem_w[s]).wait()

        def finalize(j, s):
            c = wid + j * _NW

            @pl.when((c >= 0) & (c < n_chunks))
            def _():
                for q in range(2):
                    pltpu.make_async_copy(
                        table_hbm.at[idx_v[s].at[pl.ds(q * _CH, _CH)]],
                        rows_v[s].at[pl.ds(q * _CH, _CH)], sem_g[s]).wait()
                pltpu.async_copy(rows_v[s], out_hbm.at[pl.ds(c * _CH2, _CH2)],
                                 sem_w[s])

        def half(j, s, o):
            c = wid + j * _NW

            @pl.when(c < n_chunks)
            def _():
                pltpu.make_async_copy(cid_hbm.at[pl.ds((c0 + c) * _CH2, _CH2)],
                                      idx_v[s], sem_i[s]).wait()
                for q in range(2):
                    pltpu.async_copy(
                        table_hbm.at[idx_v[s].at[pl.ds(q * _CH, _CH)]],
                        rows_v[s].at[pl.ds(q * _CH, _CH)], sem_g[s])

            finalize(j - 1, o)
            fire_idx(j + 1, o)

        fire_idx(0, 0)

        def body(jj, carry):
            j0 = 2 * jj
            wait_wb(j0 - 2, 0)
            half(j0, 0, 1)
            wait_wb(j0 - 1, 1)
            half(j0 + 1, 1, 0)
            return carry

        lax.fori_loop(0, niter, body, 0)

        finalize(2 * niter - 1, (2 * niter - 1) % 2)
        wait_wb(2 * niter - 2, 0)
        wait_wb(2 * niter - 1, 1)

    return k(table, cid)


def _edge_body(hE_ref, G_ref, B1E_ref, B2_ref, b2_ref, B3p_ref, b3p_ref,
               WV_ref, sel_ref, P_ref, EB_ref):
    bf = jnp.bfloat16
    hE = hE_ref[...].astype(bf)
    h1 = jnp.maximum(
        G_ref[...] + jnp.dot(hE, B1E_ref[...].astype(bf),
                             preferred_element_type=jnp.float32),
        0.0,
    )
    h2 = jnp.maximum(
        jnp.dot(h1.astype(bf), B2_ref[...].astype(bf),
                preferred_element_type=jnp.float32) + b2_ref[...],
        0.0,
    )
    logits = jnp.dot(h2.astype(bf), B3p_ref[...].astype(bf),
                     preferred_element_type=jnp.float32) + b3p_ref[...]
    ex = jnp.exp(logits)
    eb = jnp.dot(ex.astype(bf), sel_ref[...].astype(bf),
                 preferred_element_type=jnp.float32)
    v = jnp.dot(hE, WV_ref[...].astype(bf), preferred_element_type=jnp.float32)
    P_ref[...] = eb * v
    EB_ref[...] = eb


def _edge_compute(h_E, G, B1E, B2, b2, B3p, b3p, W_V, sel, block, blk0):
    din = h_E.shape[1]
    e, h = G.shape
    return pl.pallas_call(
        _edge_body,
        grid=(e // block,),
        in_specs=[
            pl.BlockSpec((block, din), lambda i: (i + blk0, 0)),
            pl.BlockSpec((block, h), lambda i: (i, 0)),
            pl.BlockSpec((din, h), lambda i: (0, 0)),
            pl.BlockSpec((h, h), lambda i: (0, 0)),
            pl.BlockSpec((1, h), lambda i: (0, 0)),
            pl.BlockSpec((h, h), lambda i: (0, 0)),
            pl.BlockSpec((1, h), lambda i: (0, 0)),
            pl.BlockSpec((din, h), lambda i: (0, 0)),
            pl.BlockSpec((h, h), lambda i: (0, 0)),
        ],
        out_specs=[
            pl.BlockSpec((block, h), lambda i: (i, 0)),
            pl.BlockSpec((block, h), lambda i: (i, 0)),
        ],
        out_shape=[
            jax.ShapeDtypeStruct((e, h), jnp.float32),
            jax.ShapeDtypeStruct((e, h), jnp.float32),
        ],
    )(h_E, G, B1E, B2, b2, B3p, b3p, W_V, sel)


def _sc_scatter_both(P, EB, cid, init, n_pad, rpt, c0):
    d = P.shape[1]
    n_chunks = P.shape[0] // _CH
    cpw = -(-n_chunks // _NS)
    niter = -(-cpw // 3)
    mesh = plsc.VectorSubcoreMesh(core_axis_name="c", subcore_axis_name="s")

    @functools.partial(
        pl.kernel,
        out_type=jax.ShapeDtypeStruct((_NC, n_pad, d), jnp.float32),
        mesh=mesh,
        scratch_types=[
            pltpu.VMEM((_CH,), jnp.int32),
            pltpu.VMEM((_CH,), jnp.int32),
            pltpu.VMEM((_CH,), jnp.int32),
            pltpu.VMEM((_CH, d), jnp.float32),
            pltpu.VMEM((_CH, d), jnp.float32),
            pltpu.VMEM((_CH, d), jnp.float32),
            pltpu.VMEM_SHARED((n_pad, d), jnp.float32),
            pltpu.SemaphoreType.DMA,
            pltpu.SemaphoreType.DMA,
            pltpu.SemaphoreType.DMA,
            pltpu.SemaphoreType.DMA,
            pltpu.SemaphoreType.DMA,
            pltpu.SemaphoreType.DMA,
        ],
    )
    def k(P_hbm, EB_hbm, cid_hbm, z_hbm, out_hbm,
          idx0, idx1, idx2, rows0, rows1, rows2, acc_sh,
          si0, si1, si2, sr0, sr1, sr2):
        cc = lax.axis_index("c")
        sid = lax.axis_index("s")
        idx_v = (idx0, idx1, idx2)
        rows_v = (rows0, rows1, rows2)
        sem_i = (si0, si1, si2)
        sem_r = (sr0, sr1, sr2)

        pltpu.sync_copy(z_hbm.at[cc, pl.ds(sid * rpt, rpt)],
                        acc_sh.at[pl.ds(sid * rpt, rpt)])
        plsc.subcore_barrier()

        def stream(rows_hbm):
            def fire_loads(j, s):
                c = sid + j * _NS

                @pl.when(c < n_chunks)
                def _():
                    pltpu.async_copy(cid_hbm.at[pl.ds((c0 + c) * _CH, _CH)],
                                     idx_v[s], sem_i[s])
                    pltpu.async_copy(rows_hbm.at[pl.ds(c * _CH, _CH)],
                                     rows_v[s], sem_r[s])

            def step(j, s):
                c = sid + j * _NS
                fire_loads(j + 2, (s + 2) % 3)

                @pl.when(c < n_chunks)
                def _():
                    pltpu.make_async_copy(cid_hbm.at[pl.ds((c0 + c) * _CH, _CH)],
                                          idx_v[s], sem_i[s]).wait()
                    pltpu.make_async_copy(rows_hbm.at[pl.ds(c * _CH, _CH)],
                                          rows_v[s], sem_r[s]).wait()
                    pltpu.sync_copy(rows_v[s], acc_sh.at[idx_v[s]], add=True)

            fire_loads(0, 0)
            fire_loads(1, 1)

            def body(jj, carry):
                j0 = 3 * jj
                step(j0, 0)
                step(j0 + 1, 1)
                step(j0 + 2, 2)
                return carry

            lax.fori_loop(0, niter, body, 0)

        @pl.when(cc == 0)
        def _():
            stream(P_hbm)

        @pl.when(cc == 1)
        def _():
            stream(EB_hbm)

        plsc.subcore_barrier()
        pltpu.sync_copy(acc_sh.at[pl.ds(sid * rpt, rpt)],
                        out_hbm.at[cc, pl.ds(sid * rpt, rpt)])

    return k(P, EB, cid, init)


def _finish_body(SP_ref, SE_ref, WO_ref, out_ref):
    se = SE_ref[...]
    h_agg = SP_ref[...] / jnp.where(se > 0.0, se, 1.0)
    out_ref[...] = jnp.dot(h_agg, WO_ref[...],
                           preferred_element_type=jnp.float32)


def _finish(S, W_O, n, block):
    h = W_O.shape[0]
    return pl.pallas_call(
        _finish_body,
        grid=(n // block,),
        in_specs=[
            pl.BlockSpec((None, block, h), lambda i: (0, i, 0)),
            pl.BlockSpec((None, block, h), lambda i: (1, i, 0)),
            pl.BlockSpec((h, h), lambda i: (0, 0)),
        ],
        out_specs=pl.BlockSpec((block, h), lambda i: (i, 0)),
        out_shape=jax.ShapeDtypeStruct((n, h), jnp.float32),
    )(S, S, W_O)


def kernel(h_V, h_E, center_id, batch_id, W_V, W_O,
           B1_w, B1_b, B2_w, B2_b, B3_w, B3_b):
    n, h = h_V.shape
    e, din = h_E.shape
    nh = B3_w.shape[1]
    dh = h // nh
    scale = 1.0 / np.sqrt(dh)

    B1V = B1_w[:h]
    B1E = B1_w[h:]
    b1 = B1_b.reshape(1, h)
    b2 = B2_b.reshape(1, h)
    B3p = jnp.zeros((h, h), jnp.float32).at[:, :nh].set(B3_w * scale)
    b3p = jnp.zeros((1, h), jnp.float32).at[0, :nh].set(B3_b * scale)
    sel_np = np.zeros((h, h), np.float32)
    for head in range(nh):
        sel_np[head, head * dh:(head + 1) * dh] = 1.0
    sel = jnp.asarray(sel_np)

    rpt = -(-n // (_NS * 8)) * 8
    n_pad = rpt * _NS
    zeros = jnp.zeros((_NC, n_pad, h), jnp.float32)

    blk = 1280
    nch1 = (e // _CH2 // 2) // 5 * 5
    e1 = nch1 * _CH2
    nch2 = e // _CH2 - nch1

    A = _node_prep(h_V, B1V, b1, block=1000)
    G1 = _sc_gather(A, center_id, 0, nch1)
    G2 = _sc_gather(A, center_id, nch1, nch2)
    P1, EB1 = _edge_compute(h_E, G1, B1E, B2_w, b2, B3p, b3p, W_V, sel,
                            block=blk, blk0=0)
    P2, EB2 = _edge_compute(h_E, G2, B1E, B2_w, b2, B3p, b3p, W_V, sel,
                            block=blk, blk0=e1 // blk)
    S1 = _sc_scatter_both(P1, EB1, center_id, zeros, n_pad, rpt, 0)
    S = _sc_scatter_both(P2, EB2, center_id, S1, n_pad, rpt, e1 // _CH)
    return _finish(S, W_O, n, block=1000)


# speedup vs baseline: 53.5388x; 1.0158x over previous
import functools

import jax
import jax.numpy as jnp
import numpy as np
from jax import lax
from jax.experimental import pallas as pl
from jax.experimental.pallas import tpu as pltpu
from jax.experimental.pallas import tpu_sc as plsc

_NC = 2
_NS = 16
_NW = _NC * _NS

_CH = 128


def _node_prep_body(hV_ref, W_ref, b_ref, out_ref):
    out_ref[...] = (
        jnp.dot(hV_ref[...].astype(jnp.bfloat16),
                W_ref[...].astype(jnp.bfloat16),
                preferred_element_type=jnp.float32)
        + b_ref[...]
    )


def _node_prep(h_V, B1V, b1, block):
    n, h = h_V.shape
    return pl.pallas_call(
        _node_prep_body,
        grid=(n // block,),
        in_specs=[
            pl.BlockSpec((block, h), lambda i: (i, 0)),
            pl.BlockSpec((h, h), lambda i: (0, 0)),
            pl.BlockSpec((1, h), lambda i: (0, 0)),
        ],
        out_specs=pl.BlockSpec((block, h), lambda i: (i, 0)),
        out_shape=jax.ShapeDtypeStruct((n, h), jnp.float32),
    )(h_V, B1V, b1)


_CH2 = 256


def _sc_gather(table, cid, c0, n_chunks):
    d = table.shape[1]
    cpw = -(-n_chunks // _NW)
    niter = (cpw + 1) // 2
    mesh = plsc.VectorSubcoreMesh(core_axis_name="c", subcore_axis_name="s")

    @functools.partial(
        pl.kernel,
        out_type=jax.ShapeDtypeStruct((n_chunks * _CH2, d), jnp.float32),
        mesh=mesh,
        scratch_types=[
            pltpu.VMEM((_CH2,), jnp.int32),
            pltpu.VMEM((_CH2,), jnp.int32),
            pltpu.VMEM((_CH2, d), jnp.float32),
            pltpu.VMEM((_CH2, d), jnp.float32),
            pltpu.SemaphoreType.DMA,
            pltpu.SemaphoreType.DMA,
            pltpu.SemaphoreType.DMA,
            pltpu.SemaphoreType.DMA,
            pltpu.SemaphoreType.DMA,
            pltpu.SemaphoreType.DMA,
        ],
    )
    def k(table_hbm, cid_hbm, out_hbm,
          idx0, idx1, rows0, rows1, si0, si1, sg0, sg1, sw0, sw1):
        wid = lax.axis_index("s") * _NC + lax.axis_index("c")
        idx_v = (idx0, idx1)
        rows_v = (rows0, rows1)
        sem_i = (si0, si1)
        sem_g = (sg0, sg1)
        sem_w = (sw0, sw1)

        def fire_idx(j, s):
            c = wid + j * _NW

            @pl.when(c < n_chunks)
            def _():
                pltpu.async_copy(cid_hbm.at[pl.ds((c0 + c) * _CH2, _CH2)],
                                 idx_v[s], sem_i[s])

        def wait_wb(j, s):
            c = wid + j * _NW

            @pl.when((c >= 0) & (c < n_chunks))
            def _():
                pltpu.make_async_copy(
                    rows_v[s], out_hbm.at[pl.ds(c * _CH2, _CH2)],
                    sem_w[s]).wait()

        def finalize(j, s):
            c = wid + j * _NW

            @pl.when((c >= 0) & (c < n_chunks))
            def _():
                for q in range(2):
                    pltpu.make_async_copy(
                        table_hbm.at[idx_v[s].at[pl.ds(q * _CH, _CH)]],
                        rows_v[s].at[pl.ds(q * _CH, _CH)], sem_g[s]).wait()
                pltpu.async_copy(rows_v[s], out_hbm.at[pl.ds(c * _CH2, _CH2)],
                                 sem_w[s])

        def half(j, s, o):
            c = wid + j * _NW

            @pl.when(c < n_chunks)
            def _():
                pltpu.make_async_copy(cid_hbm.at[pl.ds((c0 + c) * _CH2, _CH2)],
                                      idx_v[s], sem_i[s]).wait()
                for q in range(2):
                    pltpu.async_copy(
                        table_hbm.at[idx_v[s].at[pl.ds(q * _CH, _CH)]],
                        rows_v[s].at[pl.ds(q * _CH, _CH)], sem_g[s])

            finalize(j - 1, o)
            fire_idx(j + 1, o)

        fire_idx(0, 0)

        def body(jj, carry):
            j0 = 2 * jj
            wait_wb(j0 - 2, 0)
            half(j0, 0, 1)
            wait_wb(j0 - 1, 1)
            half(j0 + 1, 1, 0)
            return carry

        lax.fori_loop(0, niter, body, 0)

        finalize(2 * niter - 1, (2 * niter - 1) % 2)
        wait_wb(2 * niter - 2, 0)
        wait_wb(2 * niter - 1, 1)

    return k(table, cid)


def _edge_body(hE_ref, G_ref, B1E_ref, B2_ref, b2_ref, B3p_ref, b3p_ref,
               WV_ref, sel_ref, P_ref, EB_ref):
    bf = jnp.bfloat16
    hE = hE_ref[...].astype(bf)
    h1 = jnp.maximum(
        G_ref[...] + jnp.dot(hE, B1E_ref[...].astype(bf),
                             preferred_element_type=jnp.float32),
        0.0,
    )
    h2 = jnp.maximum(
        jnp.dot(h1.astype(bf), B2_ref[...].astype(bf),
                preferred_element_type=jnp.float32) + b2_ref[...],
        0.0,
    )
    logits = jnp.dot(h2.astype(bf), B3p_ref[...].astype(bf),
                     preferred_element_type=jnp.float32) + b3p_ref[...]
    ex = jnp.exp(logits)
    eb = jnp.dot(ex.astype(bf), sel_ref[...].astype(bf),
                 preferred_element_type=jnp.float32)
    v = jnp.dot(hE, WV_ref[...].astype(bf), preferred_element_type=jnp.float32)
    P_ref[...] = eb * v
    EB_ref[...] = eb


def _edge_compute(h_E, G, B1E, B2, b2, B3p, b3p, W_V, sel, block, blk0):
    din = h_E.shape[1]
    e, h = G.shape
    return pl.pallas_call(
        _edge_body,
        grid=(e // block,),
        in_specs=[
            pl.BlockSpec((block, din), lambda i: (i + blk0, 0)),
            pl.BlockSpec((block, h), lambda i: (i, 0)),
            pl.BlockSpec((din, h), lambda i: (0, 0)),
            pl.BlockSpec((h, h), lambda i: (0, 0)),
            pl.BlockSpec((1, h), lambda i: (0, 0)),
            pl.BlockSpec((h, h), lambda i: (0, 0)),
            pl.BlockSpec((1, h), lambda i: (0, 0)),
            pl.BlockSpec((din, h), lambda i: (0, 0)),
            pl.BlockSpec((h, h), lambda i: (0, 0)),
        ],
        out_specs=[
            pl.BlockSpec((block, h), lambda i: (i, 0)),
            pl.BlockSpec((block, h), lambda i: (i, 0)),
        ],
        out_shape=[
            jax.ShapeDtypeStruct((e, h), jnp.float32),
            jax.ShapeDtypeStruct((e, h), jnp.float32),
        ],
    )(h_E, G, B1E, B2, b2, B3p, b3p, W_V, sel)


def _sc_scatter_both(P, EB, cid, init, n_pad, rpt, c0):
    d = P.shape[1]
    n_chunks = P.shape[0] // _CH
    cpw = -(-n_chunks // _NS)
    niter = -(-cpw // 3)
    mesh = plsc.VectorSubcoreMesh(core_axis_name="c", subcore_axis_name="s")

    @functools.partial(
        pl.kernel,
        out_type=jax.ShapeDtypeStruct((_NC, n_pad, d), jnp.float32),
        mesh=mesh,
        scratch_types=[
            pltpu.VMEM((_CH,), jnp.int32),
            pltpu.VMEM((_CH,), jnp.int32),
            pltpu.VMEM((_CH,), jnp.int32),
            pltpu.VMEM((_CH, d), jnp.float32),
            pltpu.VMEM((_CH, d), jnp.float32),
            pltpu.VMEM((_CH, d), jnp.float32),
            pltpu.VMEM_SHARED((n_pad, d), jnp.float32),
            pltpu.SemaphoreType.DMA,
            pltpu.SemaphoreType.DMA,
            pltpu.SemaphoreType.DMA,
            pltpu.SemaphoreType.DMA,
            pltpu.SemaphoreType.DMA,
            pltpu.SemaphoreType.DMA,
        ],
    )
    def k(P_hbm, EB_hbm, cid_hbm, z_hbm, out_hbm,
          idx0, idx1, idx2, rows0, rows1, rows2, acc_sh,
          si0, si1, si2, sr0, sr1, sr2):
        cc = lax.axis_index("c")
        sid = lax.axis_index("s")
        idx_v = (idx0, idx1, idx2)
        rows_v = (rows0, rows1, rows2)
        sem_i = (si0, si1, si2)
        sem_r = (sr0, sr1, sr2)

        pltpu.sync_copy(z_hbm.at[cc, pl.ds(sid * rpt, rpt)],
                        acc_sh.at[pl.ds(sid * rpt, rpt)])
        plsc.subcore_barrier()

        def stream(rows_hbm):
            def fire_loads(j, s):
                c = sid + j * _NS

                @pl.when(c < n_chunks)
                def _():
                    pltpu.async_copy(cid_hbm.at[pl.ds((c0 + c) * _CH, _CH)],
                                     idx_v[s], sem_i[s])
                    pltpu.async_copy(rows_hbm.at[pl.ds(c * _CH, _CH)],
                                     rows_v[s], sem_r[s])

            def step(j, s):
                c = sid + j * _NS
                fire_loads(j + 2, (s + 2) % 3)

                @pl.when(c < n_chunks)
                def _():
                    pltpu.make_async_copy(cid_hbm.at[pl.ds((c0 + c) * _CH, _CH)],
                                          idx_v[s], sem_i[s]).wait()
                    pltpu.make_async_copy(rows_hbm.at[pl.ds(c * _CH, _CH)],
                                          rows_v[s], sem_r[s]).wait()
                    pltpu.sync_copy(rows_v[s], acc_sh.at[idx_v[s]], add=True)

            fire_loads(0, 0)
            fire_loads(1, 1)

            def body(jj, carry):
                j0 = 3 * jj
                step(j0, 0)
                step(j0 + 1, 1)
                step(j0 + 2, 2)
                return carry

            lax.fori_loop(0, niter, body, 0)

        @pl.when(cc == 0)
        def _():
            stream(P_hbm)

        @pl.when(cc == 1)
        def _():
            stream(EB_hbm)

        plsc.subcore_barrier()
        pltpu.sync_copy(acc_sh.at[pl.ds(sid * rpt, rpt)],
                        out_hbm.at[cc, pl.ds(sid * rpt, rpt)])

    return k(P, EB, cid, init)


def _finish_body(SP_ref, SE_ref, WO_ref, out_ref):
    se = SE_ref[...]
    h_agg = SP_ref[...] / jnp.where(se > 0.0, se, 1.0)
    out_ref[...] = jnp.dot(h_agg, WO_ref[...],
                           preferred_element_type=jnp.float32)


def _finish(S, W_O, n, block):
    h = W_O.shape[0]
    return pl.pallas_call(
        _finish_body,
        grid=(n // block,),
        in_specs=[
            pl.BlockSpec((None, block, h), lambda i: (0, i, 0)),
            pl.BlockSpec((None, block, h), lambda i: (1, i, 0)),
            pl.BlockSpec((h, h), lambda i: (0, 0)),
        ],
        out_specs=pl.BlockSpec((block, h), lambda i: (i, 0)),
        out_shape=jax.ShapeDtypeStruct((n, h), jnp.float32),
    )(S, S, W_O)


def kernel(h_V, h_E, center_id, batch_id, W_V, W_O,
           B1_w, B1_b, B2_w, B2_b, B3_w, B3_b):
    n, h = h_V.shape
    e, din = h_E.shape
    nh = B3_w.shape[1]
    dh = h // nh
    scale = 1.0 / np.sqrt(dh)

    B1V = B1_w[:h]
    B1E = B1_w[h:]
    b1 = B1_b.reshape(1, h)
    b2 = B2_b.reshape(1, h)
    B3p = jnp.zeros((h, h), jnp.float32).at[:, :nh].set(B3_w * scale)
    b3p = jnp.zeros((1, h), jnp.float32).at[0, :nh].set(B3_b * scale)
    sel_np = np.zeros((h, h), np.float32)
    for head in range(nh):
        sel_np[head, head * dh:(head + 1) * dh] = 1.0
    sel = jnp.asarray(sel_np)

    rpt = -(-n // (_NS * 8)) * 8
    n_pad = rpt * _NS
    zeros = jnp.zeros((_NC, n_pad, h), jnp.float32)

    blk = 1280
    nsl = 4
    tot = e // _CH2
    base = (tot // nsl) // 5 * 5
    nchs = [base] * (nsl - 1) + [tot - base * (nsl - 1)]

    A = _node_prep(h_V, B1V, b1, block=1000)
    Gs = []
    c0 = 0
    for nch in nchs:
        Gs.append(_sc_gather(A, center_id, c0, nch))
        c0 += nch
    S = zeros
    c0 = 0
    for G, nch in zip(Gs, nchs):
        P, EB = _edge_compute(h_E, G, B1E, B2_w, b2, B3p, b3p, W_V, sel,
                              block=blk, blk0=c0 * _CH2 // blk)
        S = _sc_scatter_both(P, EB, center_id, S, n_pad, rpt,
                             c0 * _CH2 // _CH)
        c0 += nch
    return _finish(S, W_O, n, block=1000)


# speedup vs baseline: 53.6297x; 1.0017x over previous
import functools

import jax
import jax.numpy as jnp
import numpy as np
from jax import lax
from jax.experimental import pallas as pl
from jax.experimental.pallas import tpu as pltpu
from jax.experimental.pallas import tpu_sc as plsc

_NC = 2
_NS = 16
_NW = _NC * _NS

_CH = 128


def _node_prep_body(hV_ref, W_ref, b_ref, out_ref):
    out_ref[...] = (
        jnp.dot(hV_ref[...].astype(jnp.bfloat16),
                W_ref[...].astype(jnp.bfloat16),
                preferred_element_type=jnp.float32)
        + b_ref[...]
    )


def _node_prep(h_V, B1V, b1, block):
    n, h = h_V.shape
    return pl.pallas_call(
        _node_prep_body,
        grid=(n // block,),
        in_specs=[
            pl.BlockSpec((block, h), lambda i: (i, 0)),
            pl.BlockSpec((h, h), lambda i: (0, 0)),
            pl.BlockSpec((1, h), lambda i: (0, 0)),
        ],
        out_specs=pl.BlockSpec((block, h), lambda i: (i, 0)),
        out_shape=jax.ShapeDtypeStruct((n, h), jnp.float32),
    )(h_V, B1V, b1)


_CH2 = 256


def _sc_gather(table, cid, c0, n_chunks):
    d = table.shape[1]
    cpw = -(-n_chunks // _NW)
    niter = (cpw + 1) // 2
    mesh = plsc.VectorSubcoreMesh(core_axis_name="c", subcore_axis_name="s")

    @functools.partial(
        pl.kernel,
        out_type=jax.ShapeDtypeStruct((n_chunks * _CH2, d), jnp.float32),
        mesh=mesh,
        scratch_types=[
            pltpu.VMEM((_CH2,), jnp.int32),
            pltpu.VMEM((_CH2,), jnp.int32),
            pltpu.VMEM((_CH2, d), jnp.float32),
            pltpu.VMEM((_CH2, d), jnp.float32),
            pltpu.SemaphoreType.DMA,
            pltpu.SemaphoreType.DMA,
            pltpu.SemaphoreType.DMA,
            pltpu.SemaphoreType.DMA,
            pltpu.SemaphoreType.DMA,
            pltpu.SemaphoreType.DMA,
        ],
    )
    def k(table_hbm, cid_hbm, out_hbm,
          idx0, idx1, rows0, rows1, si0, si1, sg0, sg1, sw0, sw1):
        wid = lax.axis_index("s") * _NC + lax.axis_index("c")
        idx_v = (idx0, idx1)
        rows_v = (rows0, rows1)
        sem_i = (si0, si1)
        sem_g = (sg0, sg1)
        sem_w = (sw0, sw1)

        def fire_idx(j, s):
            c = wid + j * _NW

            @pl.when(c < n_chunks)
            def _():
                pltpu.async_copy(cid_hbm.at[pl.ds((c0 + c) * _CH2, _CH2)],
                                 idx_v[s], sem_i[s])

        def wait_wb(j, s):
            c = wid + j * _NW

            @pl.when((c >= 0) & (c < n_chunks))
            def _():
                pltpu.make_async_copy(
                    rows_v[s], out_hbm.at[pl.ds(c * _CH2, _CH2)],
                    sem_w[s]).wait()

        def finalize(j, s):
            c = wid + j * _NW

            @pl.when((c >= 0) & (c < n_chunks))
            def _():
                for q in range(2):
                    pltpu.make_async_copy(
                        table_hbm.at[idx_v[s].at[pl.ds(q * _CH, _CH)]],
                        rows_v[s].at[pl.ds(q * _CH, _CH)], sem_g[s]).wait()
                pltpu.async_copy(rows_v[s], out_hbm.at[pl.ds(c * _CH2, _CH2)],
                                 sem_w[s])

        def half(j, s, o):
            c = wid + j * _NW

            @pl.when(c < n_chunks)
            def _():
                pltpu.make_async_copy(cid_hbm.at[pl.ds((c0 + c) * _CH2, _CH2)],
                                      idx_v[s], sem_i[s]).wait()
                for q in range(2):
                    pltpu.async_copy(
                        table_hbm.at[idx_v[s].at[pl.ds(q * _CH, _CH)]],
                        rows_v[s].at[pl.ds(q * _CH, _CH)], sem_g[s])

            finalize(j - 1, o)
            fire_idx(j + 1, o)

        fire_idx(0, 0)

        def body(jj, carry):
            j0 = 2 * jj
            wait_wb(j0 - 2, 0)
            half(j0, 0, 1)
            wait_wb(j0 - 1, 1)
            half(j0 + 1, 1, 0)
            return carry

        lax.fori_loop(0, niter, body, 0)

        finalize(2 * niter - 1, (2 * niter - 1) % 2)
        wait_wb(2 * niter - 2, 0)
        wait_wb(2 * niter - 1, 1)

    return k(table, cid)


def _edge_body(hE_ref, G_ref, B1E_ref, B2_ref, b2_ref, B3p_ref, b3p_ref,
               WV_ref, sel_ref, P_ref, EB_ref):
    bf = jnp.bfloat16
    hE = hE_ref[...].astype(bf)
    h1 = jnp.maximum(
        G_ref[...] + jnp.dot(hE, B1E_ref[...].astype(bf),
                             preferred_element_type=jnp.float32),
        0.0,
    )
    h2 = jnp.maximum(
        jnp.dot(h1.astype(bf), B2_ref[...].astype(bf),
                preferred_element_type=jnp.float32) + b2_ref[...],
        0.0,
    )
    logits = jnp.dot(h2.astype(bf), B3p_ref[...].astype(bf),
                     preferred_element_type=jnp.float32) + b3p_ref[...]
    ex = jnp.exp(logits)
    eb = jnp.dot(ex.astype(bf), sel_ref[...].astype(bf),
                 preferred_element_type=jnp.float32)
    v = jnp.dot(hE, WV_ref[...].astype(bf), preferred_element_type=jnp.float32)
    P_ref[...] = eb * v
    EB_ref[...] = eb


def _edge_compute(h_E, G, B1E, B2, b2, B3p, b3p, W_V, sel, block, blk0):
    din = h_E.shape[1]
    e, h = G.shape
    return pl.pallas_call(
        _edge_body,
        grid=(e // block,),
        in_specs=[
            pl.BlockSpec((block, din), lambda i: (i + blk0, 0)),
            pl.BlockSpec((block, h), lambda i: (i, 0)),
            pl.BlockSpec((din, h), lambda i: (0, 0)),
            pl.BlockSpec((h, h), lambda i: (0, 0)),
            pl.BlockSpec((1, h), lambda i: (0, 0)),
            pl.BlockSpec((h, h), lambda i: (0, 0)),
            pl.BlockSpec((1, h), lambda i: (0, 0)),
            pl.BlockSpec((din, h), lambda i: (0, 0)),
            pl.BlockSpec((h, h), lambda i: (0, 0)),
        ],
        out_specs=[
            pl.BlockSpec((block, h), lambda i: (i, 0)),
            pl.BlockSpec((block, h), lambda i: (i, 0)),
        ],
        out_shape=[
            jax.ShapeDtypeStruct((e, h), jnp.float32),
            jax.ShapeDtypeStruct((e, h), jnp.float32),
        ],
    )(h_E, G, B1E, B2, b2, B3p, b3p, W_V, sel)


def _sc_scatter_both(P, EB, cid, init, n_pad, rpt, c0):
    d = P.shape[1]
    n_chunks = P.shape[0] // _CH
    cpw = -(-n_chunks // _NS)
    niter = -(-cpw // 3)
    mesh = plsc.VectorSubcoreMesh(core_axis_name="c", subcore_axis_name="s")

    @functools.partial(
        pl.kernel,
        out_type=jax.ShapeDtypeStruct((_NC, n_pad, d), jnp.float32),
        mesh=mesh,
        scratch_types=[
            pltpu.VMEM((_CH,), jnp.int32),
            pltpu.VMEM((_CH,), jnp.int32),
            pltpu.VMEM((_CH,), jnp.int32),
            pltpu.VMEM((_CH, d), jnp.float32),
            pltpu.VMEM((_CH, d), jnp.float32),
            pltpu.VMEM((_CH, d), jnp.float32),
            pltpu.VMEM_SHARED((n_pad, d), jnp.float32),
            pltpu.SemaphoreType.DMA,
            pltpu.SemaphoreType.DMA,
            pltpu.SemaphoreType.DMA,
            pltpu.SemaphoreType.DMA,
            pltpu.SemaphoreType.DMA,
            pltpu.SemaphoreType.DMA,
        ],
    )
    def k(P_hbm, EB_hbm, cid_hbm, z_hbm, out_hbm,
          idx0, idx1, idx2, rows0, rows1, rows2, acc_sh,
          si0, si1, si2, sr0, sr1, sr2):
        cc = lax.axis_index("c")
        sid = lax.axis_index("s")
        idx_v = (idx0, idx1, idx2)
        rows_v = (rows0, rows1, rows2)
        sem_i = (si0, si1, si2)
        sem_r = (sr0, sr1, sr2)

        pltpu.sync_copy(z_hbm.at[cc, pl.ds(sid * rpt, rpt)],
                        acc_sh.at[pl.ds(sid * rpt, rpt)])
        plsc.subcore_barrier()

        def stream(rows_hbm):
            def fire_loads(j, s):
                c = sid + j * _NS

                @pl.when(c < n_chunks)
                def _():
                    pltpu.async_copy(cid_hbm.at[pl.ds((c0 + c) * _CH, _CH)],
                                     idx_v[s], sem_i[s])
                    pltpu.async_copy(rows_hbm.at[pl.ds(c * _CH, _CH)],
                                     rows_v[s], sem_r[s])

            def step(j, s):
                c = sid + j * _NS
                fire_loads(j + 2, (s + 2) % 3)

                @pl.when(c < n_chunks)
                def _():
                    pltpu.make_async_copy(cid_hbm.at[pl.ds((c0 + c) * _CH, _CH)],
                                          idx_v[s], sem_i[s]).wait()
                    pltpu.make_async_copy(rows_hbm.at[pl.ds(c * _CH, _CH)],
                                          rows_v[s], sem_r[s]).wait()
                    pltpu.sync_copy(rows_v[s], acc_sh.at[idx_v[s]], add=True)

            fire_loads(0, 0)
            fire_loads(1, 1)

            def body(jj, carry):
                j0 = 3 * jj
                step(j0, 0)
                step(j0 + 1, 1)
                step(j0 + 2, 2)
                return carry

            lax.fori_loop(0, niter, body, 0)

        @pl.when(cc == 0)
        def _():
            stream(P_hbm)

        @pl.when(cc == 1)
        def _():
            stream(EB_hbm)

        plsc.subcore_barrier()
        pltpu.sync_copy(acc_sh.at[pl.ds(sid * rpt, rpt)],
                        out_hbm.at[cc, pl.ds(sid * rpt, rpt)])

    return k(P, EB, cid, init)


def _finish_body(SP_ref, SE_ref, WO_ref, out_ref):
    se = SE_ref[...]
    h_agg = SP_ref[...] / jnp.where(se > 0.0, se, 1.0)
    out_ref[...] = jnp.dot(h_agg, WO_ref[...],
                           preferred_element_type=jnp.float32)


def _finish(S, W_O, n, block):
    h = W_O.shape[0]
    return pl.pallas_call(
        _finish_body,
        grid=(n // block,),
        in_specs=[
            pl.BlockSpec((None, block, h), lambda i: (0, i, 0)),
            pl.BlockSpec((None, block, h), lambda i: (1, i, 0)),
            pl.BlockSpec((h, h), lambda i: (0, 0)),
        ],
        out_specs=pl.BlockSpec((block, h), lambda i: (i, 0)),
        out_shape=jax.ShapeDtypeStruct((n, h), jnp.float32),
    )(S, S, W_O)


def kernel(h_V, h_E, center_id, batch_id, W_V, W_O,
           B1_w, B1_b, B2_w, B2_b, B3_w, B3_b):
    n, h = h_V.shape
    e, din = h_E.shape
    nh = B3_w.shape[1]
    dh = h // nh
    scale = 1.0 / np.sqrt(dh)

    B1V = B1_w[:h]
    B1E = B1_w[h:]
    b1 = B1_b.reshape(1, h)
    b2 = B2_b.reshape(1, h)
    B3p = jnp.zeros((h, h), jnp.float32).at[:, :nh].set(B3_w * scale)
    b3p = jnp.zeros((1, h), jnp.float32).at[0, :nh].set(B3_b * scale)
    sel_np = np.zeros((h, h), np.float32)
    for head in range(nh):
        sel_np[head, head * dh:(head + 1) * dh] = 1.0
    sel = jnp.asarray(sel_np)

    rpt = -(-n // (_NS * 8)) * 8
    n_pad = rpt * _NS
    zeros = jnp.zeros((_NC, n_pad, h), jnp.float32)

    blk = 1280
    nsl = 3
    tot = e // _CH2
    base = (tot // nsl) // 5 * 5
    nchs = [base] * (nsl - 1) + [tot - base * (nsl - 1)]

    A = _node_prep(h_V, B1V, b1, block=1000)
    Gs = []
    c0 = 0
    for nch in nchs:
        Gs.append(_sc_gather(A, center_id, c0, nch))
        c0 += nch
    S = zeros
    c0 = 0
    for G, nch in zip(Gs, nchs):
        P, EB = _edge_compute(h_E, G, B1E, B2_w, b2, B3p, b3p, W_V, sel,
                              block=blk, blk0=c0 * _CH2 // blk)
        S = _sc_scatter_both(P, EB, center_id, S, n_pad, rpt,
                             c0 * _CH2 // _CH)
        c0 += nch
    return _finish(S, W_O, n, block=1000)
